# baseline (speedup 1.0000x reference)
import functools

import jax
import jax.numpy as jnp
from jax import lax
from jax.experimental import pallas as pl
from jax.experimental.pallas import tpu as pltpu
from jax.experimental.pallas import tpu_sc as plsc

N = 10000
F = 128
NPAD = 10112
E2 = 320000 + N
TILES = 32
CW = 96
CH0 = 112
CH1 = 112
G = 16
NR = 16 * (CH0 + CH1)
E2P = NR * CW
RPT = NPAD // 16


def _lane_bcast(v, i):
    return lax.gather(
        v,
        jnp.full((16, 1), i, jnp.int32),
        lax.GatherDimensionNumbers(
            offset_dims=(), collapsed_slice_dims=(0,), start_index_map=(0,)),
        (1,),
        mode=lax.GatherScatterMode.PROMISE_IN_BOUNDS)


def _emit_gat_head(x, w_ref, atts_ref, attd_ref, h_ref, as_ref, ad_ref, m_ref):
    h = jnp.dot(x, w_ref[...], preferred_element_type=jnp.float32)
    h_ref[...] = h
    a_s = jnp.sum(h * atts_ref[...], axis=1, keepdims=True)
    a_d = jnp.sum(h * attd_ref[...], axis=1, keepdims=True)
    as_ref[...] = a_s
    ad_ref[...] = a_d
    m_ref[...] = jnp.full((8, 128), jnp.max(a_s), jnp.float32)


def _first_body(x_ref, w_ref, atts_ref, attd_ref, h_ref, as_ref, ad_ref, m_ref):
    _emit_gat_head(x_ref[...], w_ref, atts_ref, attd_ref,
                   h_ref, as_ref, ad_ref, m_ref)


def _stage_body(agg_ref, den_ref, bias_ref, gam_ref, bet_ref, mu_ref, var_ref,
                w_ref, atts_ref, attd_ref, h_ref, as_ref, ad_ref, m_ref):
    a = agg_ref[0] + agg_ref[1]
    dn = jnp.sum(den_ref[...], axis=0)[:, None] + 1e-16
    x = jnp.maximum(a / dn + bias_ref[...], 0.0)
    x = (x - mu_ref[...]) * (gam_ref[...] * lax.rsqrt(var_ref[...] + 1e-5)) \
        + bet_ref[...]
    rows = lax.broadcasted_iota(jnp.int32, (NPAD, 1), 0)
    x = jnp.where(rows < N, x, 0.0)
    _emit_gat_head(x, w_ref, atts_ref, attd_ref, h_ref, as_ref, ad_ref, m_ref)


def _final_body(agg_ref, den_ref, bias_ref, out_ref):
    a = agg_ref[0] + agg_ref[1]
    dn = jnp.sum(den_ref[...], axis=0)[:, None] + 1e-16
    out_ref[...] = jnp.maximum(a / dn + bias_ref[...], 0.0)


_HEAD_OUT = [
    jax.ShapeDtypeStruct((NPAD, F), jnp.float32),
    jax.ShapeDtypeStruct((NPAD, 1), jnp.float32),
    jax.ShapeDtypeStruct((NPAD, 1), jnp.float32),
    jax.ShapeDtypeStruct((8, 128), jnp.float32),
]

_tc_first = pl.pallas_call(_first_body, out_shape=_HEAD_OUT)
_tc_stage = pl.pallas_call(_stage_body, out_shape=_HEAD_OUT)
_tc_final = pl.pallas_call(
    _final_body, out_shape=jax.ShapeDtypeStruct((NPAD, F), jnp.float32))


_mesh = plsc.VectorSubcoreMesh(core_axis_name="c", subcore_axis_name="s")


@functools.partial(
    pl.kernel,
    out_type=[
        jax.ShapeDtypeStruct((2, NPAD, F), jnp.float32),
        jax.ShapeDtypeStruct((TILES * NPAD,), jnp.float32),
    ],
    mesh=_mesh,
    compiler_params=pltpu.CompilerParams(needs_layout_passes=False),
    scratch_types=[
        [pltpu.VMEM((G, CW), jnp.int32) for _ in range(2)],
        [pltpu.VMEM((G, CW), jnp.int32) for _ in range(2)],
        pltpu.VMEM((16,), jnp.float32),
        pltpu.VMEM((NPAD,), jnp.float32),
        [pltpu.VMEM((CW,), jnp.float32) for _ in range(2)],
        [pltpu.VMEM((CW,), jnp.float32) for _ in range(2)],
        pltpu.VMEM((CW,), jnp.float32),
        [pltpu.VMEM((CW, F), jnp.float32) for _ in range(2)],
        pltpu.VMEM_SHARED((NPAD, F), jnp.float32),
        [pltpu.SemaphoreType.DMA for _ in range(2)],
        [pltpu.SemaphoreType.DMA for _ in range(2)],
    ],
)
def _sc_edge(src_hbm, dst_hbm, as_hbm, ad_hbm, m_hbm, h_hbm,
             agg_out, den_out,
             srcw, dstw, mv, denv, ga, gb, eb, rb,
             aggsh, sg, sw):
    cid = lax.axis_index("c")
    sid = lax.axis_index("s")
    wid = cid * 16 + sid
    base_row = jnp.where(cid == 0, sid * CH0, 16 * CH0 + sid * CH1)
    n_win = jnp.where(cid == 0, CH0 // G, CH1 // G)

    pltpu.sync_copy(m_hbm, mv)

    zero16 = jnp.zeros((16,), jnp.float32)

    def _zd(i, c):
        denv[pl.ds(i * 16, 16)] = zero16
        return c

    lax.fori_loop(0, NPAD // 16, _zd, 0)

    def _zr(r, c):
        for v in range(F // 16):
            rb[0][r, pl.ds(v * 16, 16)] = zero16
        return c

    lax.fori_loop(0, CW, _zr, 0)

    base = sid * RPT
    for k in range(6):
        pltpu.sync_copy(rb[0], aggsh.at[pl.ds(base + k * CW, CW)])
    pltpu.sync_copy(rb[0].at[pl.ds(0, RPT - 6 * CW)],
                    aggsh.at[pl.ds(base + 6 * CW, RPT - 6 * CW)])
    plsc.subcore_barrier()

    Mv = mv[...]

    def _win_fetch(w, wk):
        rs = pl.ds(base_row + w * G, G)
        pltpu.async_copy(src_hbm.at[rs], srcw[wk], sw[wk])
        pltpu.async_copy(dst_hbm.at[rs], dstw[wk], sw[wk])

    def _win_wait(w, wk):
        rs = pl.ds(base_row + w * G, G)
        pltpu.make_async_copy(src_hbm.at[rs], srcw[wk], sw[wk]).wait()
        pltpu.make_async_copy(dst_hbm.at[rs], dstw[wk], sw[wk]).wait()

    def _fetch(wk, g, k):
        pltpu.async_copy(h_hbm.at[srcw[wk].at[g]], rb[k], sg[k])
        pltpu.async_copy(as_hbm.at[srcw[wk].at[g]], ga[k], sg[k])
        pltpu.async_copy(ad_hbm.at[dstw[wk].at[g]], gb[k], sg[k])

    def _gath_wait(wk, g, k):
        pltpu.make_async_copy(h_hbm.at[srcw[wk].at[g]], rb[k], sg[k]).wait()
        pltpu.make_async_copy(as_hbm.at[srcw[wk].at[g]], ga[k], sg[k]).wait()
        pltpu.make_async_copy(ad_hbm.at[dstw[wk].at[g]], gb[k], sg[k]).wait()

    def _process(wk, g, k):
        for gg in range(CW // 16):
            sl = pl.ds(gg * 16, 16)
            va = ga[k][sl]
            vb = gb[k][sl]
            d = dstw[wk][g, sl]
            t = va + vb
            t = jnp.maximum(t, 0.2 * t)
            c = vb + Mv
            c = jnp.maximum(c, 0.2 * c)
            e = jnp.exp(t - c)
            eb[sl] = e
            plsc.addupdate_scatter(denv, [d], e)

        def _grp(gg, c):
            evec = eb[pl.ds(gg * 16, 16)]
            for i in range(16):
                r = gg * 16 + i
                ebc = _lane_bcast(evec, i)
                for v in range(F // 16):
                    sl = pl.ds(v * 16, 16)
                    rb[k][r, sl] = rb[k][r, sl] * ebc
            return c

        lax.fori_loop(0, CW // 16, _grp, 0)
        pltpu.sync_copy(rb[k], aggsh.at[dstw[wk].at[g]], add=True)

    def _window(w, wk):
        _win_wait(w, wk)
        _win_fetch(w + 1, 1 - wk)
        _fetch(wk, 0, 0)

        def _pairs(p, c):
            g0 = 2 * p
            _fetch(wk, g0 + 1, 1)
            _gath_wait(wk, g0, 0)
            _process(wk, g0, 0)

            @pl.when(g0 + 2 < G)
            def _():
                _fetch(wk, g0 + 2, 0)

            _gath_wait(wk, g0 + 1, 1)
            _process(wk, g0 + 1, 1)
            return c

        lax.fori_loop(0, G // 2, _pairs, 0)

    _win_fetch(0, 0)

    def _wloop(w, c):
        for wk in range(2):
            @pl.when(lax.rem(w, 2) == wk)
            def _(wk=wk):
                _window(w, wk)
        return c

    lax.fori_loop(0, n_win, _wloop, 0)
    for wk in range(2):
        @pl.when(lax.rem(n_win, 2) == wk)
        def _(wk=wk):
            _win_wait(n_win, wk)

    plsc.subcore_barrier()
    for k in range(6):
        pltpu.sync_copy(aggsh.at[pl.ds(base + k * CW, CW)],
                        agg_out.at[cid, pl.ds(base + k * CW, CW)])
    rem = RPT - 6 * CW
    pltpu.sync_copy(aggsh.at[pl.ds(base + 6 * CW, rem)],
                    agg_out.at[cid, pl.ds(base + 6 * CW, rem)])
    pltpu.sync_copy(denv, den_out.at[pl.ds(wid * NPAD, NPAD)])


def _layer_inputs(p):
    return (p["W"], p["att_src"].reshape(1, F), p["att_dst"].reshape(1, F))


def kernel(feat, edge_index, params):
    x = jnp.pad(feat, ((0, NPAD - N), (0, 0)))
    loop = jnp.arange(N, dtype=edge_index.dtype)
    src = jnp.concatenate([edge_index[0], loop])
    dst = jnp.concatenate([edge_index[1], loop])
    src3 = jnp.pad(src, (0, (NR + G) * CW - E2)).reshape(NR + G, CW)
    dst3 = jnp.pad(dst, (0, (NR + G) * CW - E2),
                   constant_values=N).reshape(NR + G, CW)

    def edge_phase(h, a_s, a_d, m8):
        agg, den = _sc_edge(src3, dst3, a_s.reshape(NPAD), a_d.reshape(NPAD),
                            m8[0, :16], h)
        return agg, den.reshape(TILES, NPAD)

    p = params
    h, a_s, a_d, m8 = _tc_first(x, *_layer_inputs(p["gc1e"]))
    agg, den = edge_phase(h, a_s, a_d, m8)
    for prev, bn, cur in (("gc1e", "bn1e", "gc2e"),
                          ("gc2e", "bn2e", "gc1d"),
                          ("gc1d", "bn1d", "gc2d")):
        b = p[bn]
        h, a_s, a_d, m8 = _tc_stage(
            agg, den, p[prev]["bias"].reshape(1, F),
            b["gamma"].reshape(1, F), b["beta"].reshape(1, F),
            b["mean"].reshape(1, F), b["var"].reshape(1, F),
            *_layer_inputs(p[cur]))
        agg, den = edge_phase(h, a_s, a_d, m8)
    out = _tc_final(agg, den, p["gc2d"]["bias"].reshape(1, F))
    return out[:N]


# speedup vs baseline: 2.3805x; 2.3805x over previous
import functools

import jax
import jax.numpy as jnp
from jax import lax
from jax.experimental import pallas as pl
from jax.experimental.pallas import tpu as pltpu
from jax.experimental.pallas import tpu_sc as plsc

N = 10000
F = 128
NPAD = 10112
E2 = 320000 + N
TILES = 32
CW = 64
CH0 = 208
CH1 = 116
CHA = 210
RPT = NPAD // 16


def _lane_bcast(v, i):
    return lax.gather(
        v,
        jnp.full((16, 1), i, jnp.int32),
        lax.GatherDimensionNumbers(
            offset_dims=(), collapsed_slice_dims=(0,), start_index_map=(0,)),
        (1,),
        mode=lax.GatherScatterMode.PROMISE_IN_BOUNDS)


def _emit_gat_head(x, w_ref, atts_ref, attd_ref, h_ref, as_ref, ad_ref, m_ref):
    h = jnp.dot(x, w_ref[...], preferred_element_type=jnp.float32)
    h_ref[...] = h
    a_s = jnp.sum(h * atts_ref[...], axis=1, keepdims=True)
    a_d = jnp.sum(h * attd_ref[...], axis=1, keepdims=True)
    as_ref[...] = a_s
    ad_ref[...] = a_d
    m_ref[...] = jnp.full((8, 128), jnp.max(a_s), jnp.float32)


def _first_body(x_ref, w_ref, atts_ref, attd_ref, h_ref, as_ref, ad_ref, m_ref):
    _emit_gat_head(x_ref[...], w_ref, atts_ref, attd_ref,
                   h_ref, as_ref, ad_ref, m_ref)


def _stage_body(agg_ref, den_ref, bias_ref, gam_ref, bet_ref, mu_ref, var_ref,
                w_ref, atts_ref, attd_ref, h_ref, as_ref, ad_ref, m_ref):
    a = agg_ref[0] + agg_ref[1]
    dn = jnp.sum(den_ref[...], axis=0)[:, None] + 1e-16
    x = jnp.maximum(a / dn + bias_ref[...], 0.0)
    x = (x - mu_ref[...]) * (gam_ref[...] * lax.rsqrt(var_ref[...] + 1e-5)) \
        + bet_ref[...]
    rows = lax.broadcasted_iota(jnp.int32, (NPAD, 1), 0)
    x = jnp.where(rows < N, x, 0.0)
    _emit_gat_head(x, w_ref, atts_ref, attd_ref, h_ref, as_ref, ad_ref, m_ref)


def _final_body(agg_ref, den_ref, bias_ref, out_ref):
    a = agg_ref[0] + agg_ref[1]
    dn = jnp.sum(den_ref[...], axis=0)[:, None] + 1e-16
    out_ref[...] = jnp.maximum(a / dn + bias_ref[...], 0.0)


_HEAD_OUT = [
    jax.ShapeDtypeStruct((NPAD, F), jnp.float32),
    jax.ShapeDtypeStruct((NPAD, 1), jnp.float32),
    jax.ShapeDtypeStruct((NPAD, 1), jnp.float32),
    jax.ShapeDtypeStruct((8, 128), jnp.float32),
]

_tc_first = pl.pallas_call(_first_body, out_shape=_HEAD_OUT)
_tc_stage = pl.pallas_call(_stage_body, out_shape=_HEAD_OUT)
_tc_final = pl.pallas_call(
    _final_body, out_shape=jax.ShapeDtypeStruct((NPAD, F), jnp.float32))


_mesh = plsc.VectorSubcoreMesh(core_axis_name="c", subcore_axis_name="s")


@functools.partial(
    pl.kernel,
    out_type=[
        jax.ShapeDtypeStruct((2, NPAD, F), jnp.float32),
        jax.ShapeDtypeStruct((2 * NPAD,), jnp.float32),
    ],
    mesh=_mesh,
    compiler_params=pltpu.CompilerParams(needs_layout_passes=False),
    scratch_types=[
        pltpu.VMEM((CHA * CW,), jnp.int32),
        pltpu.VMEM((CHA * CW,), jnp.int32),
        pltpu.VMEM((8, CW), jnp.int32),
        pltpu.VMEM((16,), jnp.float32),
        [pltpu.VMEM((CW,), jnp.float32) for _ in range(2)],
        [pltpu.VMEM((CW,), jnp.float32) for _ in range(2)],
        pltpu.VMEM((CW,), jnp.float32),
        [pltpu.VMEM((CW, F), jnp.float32) for _ in range(2)],
        pltpu.VMEM_SHARED((NPAD, F), jnp.float32),
        pltpu.VMEM_SHARED((NPAD,), jnp.float32),
        [pltpu.SemaphoreType.DMA for _ in range(2)],
    ],
)
def _sc_edge(src_hbm, dst_hbm, as_hbm, ad_hbm, m_hbm, zero_hbm, h_hbm,
             agg_out, den_out,
             srcv, dstv, dsts, mv, ga, gb, eb, rb,
             aggsh, densh, sg):
    cid = lax.axis_index("c")
    sid = lax.axis_index("s")
    wid = cid * 16 + sid
    ch_real = jnp.where(cid == 0, CH0, CH1)

    pltpu.sync_copy(src_hbm.at[pl.ds(wid * CHA * CW, CHA * CW)], srcv)
    pltpu.sync_copy(dst_hbm.at[pl.ds(wid * CHA * CW, CHA * CW)], dstv)
    pltpu.sync_copy(m_hbm, mv)

    zero16 = jnp.zeros((16,), jnp.float32)

    def _zr(r, c):
        for v in range(F // 16):
            rb[0][r, pl.ds(v * 16, 16)] = zero16
        return c

    lax.fori_loop(0, CW, _zr, 0)

    base = sid * RPT
    for k in range(6):
        pltpu.sync_copy(rb[0], aggsh.at[pl.ds(base + k * CW, CW)])
    pltpu.sync_copy(rb[0].at[pl.ds(0, RPT - 6 * CW)],
                    aggsh.at[pl.ds(base + 6 * CW, RPT - 6 * CW)])

    @pl.when(sid == 0)
    def _():
        pltpu.sync_copy(zero_hbm, densh)

    plsc.subcore_barrier()

    Mv = mv[...]

    def _fetch(j, k):
        pltpu.async_copy(h_hbm.at[srcv.at[pl.ds(j * CW, CW)]], rb[k], sg[k])
        pltpu.async_copy(as_hbm.at[srcv.at[pl.ds(j * CW, CW)]], ga[k], sg[k])
        pltpu.async_copy(ad_hbm.at[dstv.at[pl.ds(j * CW, CW)]], gb[k], sg[k])

    def _gath_wait(j, k):
        pltpu.make_async_copy(h_hbm.at[srcv.at[pl.ds(j * CW, CW)]],
                              rb[k], sg[k]).wait()
        pltpu.make_async_copy(as_hbm.at[srcv.at[pl.ds(j * CW, CW)]],
                              ga[k], sg[k]).wait()
        pltpu.make_async_copy(ad_hbm.at[dstv.at[pl.ds(j * CW, CW)]],
                              gb[k], sg[k]).wait()

    def _process(j, k):
        for g in range(CW // 16):
            sl = pl.ds(g * 16, 16)
            va = ga[k][sl]
            vb = gb[k][sl]
            d = dstv[pl.ds(j * CW + g * 16, 16)]
            dsts[k, sl] = d
            t = va + vb
            t = jnp.maximum(t, 0.2 * t)
            c = vb + Mv
            c = jnp.maximum(c, 0.2 * c)
            eb[sl] = jnp.exp(t - c)
        pltpu.sync_copy(eb, densh.at[dsts.at[k]], add=True)

        def _grp(g, c):
            evec = eb[pl.ds(g * 16, 16)]
            for i in range(16):
                r = g * 16 + i
                ebc = _lane_bcast(evec, i)
                for v in range(F // 16):
                    sl = pl.ds(v * 16, 16)
                    rb[k][r, sl] = rb[k][r, sl] * ebc
            return c

        lax.fori_loop(0, CW // 16, _grp, 0)
        pltpu.sync_copy(rb[k], aggsh.at[dsts.at[k]], add=True)

    _fetch(0, 0)

    def _pair(p, c):
        j0 = 2 * p
        _fetch(j0 + 1, 1)
        _gath_wait(j0, 0)
        _process(j0, 0)
        _fetch(j0 + 2, 0)
        _gath_wait(j0 + 1, 1)
        _process(j0 + 1, 1)
        return c

    lax.fori_loop(0, ch_real // 2, _pair, 0)
    _gath_wait(ch_real, 0)

    plsc.subcore_barrier()
    for k in range(6):
        pltpu.sync_copy(aggsh.at[pl.ds(base + k * CW, CW)],
                        agg_out.at[cid, pl.ds(base + k * CW, CW)])
    rem = RPT - 6 * CW
    pltpu.sync_copy(aggsh.at[pl.ds(base + 6 * CW, rem)],
                    agg_out.at[cid, pl.ds(base + 6 * CW, rem)])

    @pl.when(sid == 0)
    def _():
        pltpu.sync_copy(densh, den_out.at[pl.ds(cid * NPAD, NPAD)])


def _layer_inputs(p):
    return (p["W"], p["att_src"].reshape(1, F), p["att_dst"].reshape(1, F))


def kernel(feat, edge_index, params):
    x = jnp.pad(feat, ((0, NPAD - N), (0, 0)))
    loop = jnp.arange(N, dtype=edge_index.dtype)
    src = jnp.concatenate([edge_index[0], loop])
    dst = jnp.concatenate([edge_index[1], loop])

    def tile_layout(flat, padval):
        e0, e1 = 16 * CH0 * CW, 16 * CH1 * CW
        seq = jnp.pad(flat, (0, e0 + e1 - E2), constant_values=padval)
        b0 = jnp.pad(seq[:e0].reshape(16, CH0 * CW),
                     ((0, 0), (0, (CHA - CH0) * CW)), constant_values=padval)
        b1 = jnp.pad(seq[e0:].reshape(16, CH1 * CW),
                     ((0, 0), (0, (CHA - CH1) * CW)), constant_values=padval)
        return jnp.concatenate([b0, b1], axis=0).reshape(-1)

    src3 = tile_layout(src, 0)
    dst3 = tile_layout(dst, N)
    zero_n = jnp.zeros((NPAD,), jnp.float32)

    def edge_phase(h, a_s, a_d, m8):
        agg, den = _sc_edge(src3, dst3, a_s.reshape(NPAD), a_d.reshape(NPAD),
                            m8[0, :16], zero_n, h)
        return agg, den.reshape(2, NPAD)

    p = params
    h, a_s, a_d, m8 = _tc_first(x, *_layer_inputs(p["gc1e"]))
    agg, den = edge_phase(h, a_s, a_d, m8)
    for prev, bn, cur in (("gc1e", "bn1e", "gc2e"),
                          ("gc2e", "bn2e", "gc1d"),
                          ("gc1d", "bn1d", "gc2d")):
        b = p[bn]
        h, a_s, a_d, m8 = _tc_stage(
            agg, den, p[prev]["bias"].reshape(1, F),
            b["gamma"].reshape(1, F), b["beta"].reshape(1, F),
            b["mean"].reshape(1, F), b["var"].reshape(1, F),
            *_layer_inputs(p[cur]))
        agg, den = edge_phase(h, a_s, a_d, m8)
    out = _tc_final(agg, den, p["gc2d"]["bias"].reshape(1, F))
    return out[:N]
